# Initial kernel scaffold; baseline (speedup 1.0000x reference)
import jax
import jax.numpy as jnp
from jax.experimental import pallas as pl


def kernel(x, conv1_w, conv1_b, bn1_g, bn1_b, conv2_w, conv2_b, bn2_g, bn2_b, ca_w1, ca_w2, sa_w, ds_w):
    raise NotImplementedError("write your pallas kernel here")


# speedup vs baseline: 1.9521x; 1.9521x over previous
import functools

import jax
import jax.numpy as jnp
from jax.experimental import pallas as pl
from jax.experimental.pallas import tpu as pltpu

_VMEM_LIMIT = 48 * 1024 * 1024


def _conv_acc(xv, wv, ho, wo, kh, kw):
    n = xv.shape[0]
    cin = xv.shape[3]
    cblk = wv.shape[3]
    acc = jnp.zeros((n * ho * wo, cblk), jnp.float32)
    for dh in range(kh):
        for dw in range(kw):
            lhs = xv[:, dh:dh + ho, dw:dw + wo, :].reshape(n * ho * wo, cin)
            acc = acc + jnp.dot(lhs, wv[dh, dw],
                                preferred_element_type=jnp.float32)
    return acc


def _bn_affine(y, g, be, cnt, eps):
    mean = jnp.sum(y, axis=0) / cnt
    var = jnp.sum(y * y, axis=0) / cnt - mean * mean
    scale = g.astype(jnp.float32) * jax.lax.rsqrt(var + eps)
    shift = be.astype(jnp.float32) - mean * scale
    return y * scale + shift


def _k1_body(x_ref, w1_ref, b1_ref, g1_ref, be1_ref, dsw_ref,
             t1_ref, res_ref, *, eps):
    _, n, h1, w1o, cblk = t1_ref.shape
    ho, wo = res_ref.shape[2], res_ref.shape[3]
    xv = x_ref[...]

    y = _conv_acc(xv, w1_ref[0], h1, w1o, 3, 3) \
        + b1_ref[0, 0].astype(jnp.float32)
    t1 = jnp.maximum(
        _bn_affine(y, g1_ref[0, 0], be1_ref[0, 0], float(n * h1 * w1o), eps),
        0.0)
    t1_ref[...] = t1.reshape(1, n, h1, w1o, cblk).astype(t1_ref.dtype)

    res = _conv_acc(xv, dsw_ref[0], ho, wo, 5, 5)
    res_ref[...] = res.reshape(1, n, ho, wo, cblk)


def _k2_body(t1_ref, w2_ref, b2_ref, g2_ref, be2_ref,
             y_ref, avg_ref, max_ref, *, eps):
    _, n, ho, wo, cblk = y_ref.shape
    tv = jnp.concatenate([t1_ref[0], t1_ref[1]], axis=-1)

    y = _conv_acc(tv, w2_ref[0], ho, wo, 3, 3) \
        + b2_ref[0, 0].astype(jnp.float32)
    yb = _bn_affine(y, g2_ref[0, 0], be2_ref[0, 0], float(n * ho * wo), eps)

    y3 = yb.reshape(n, ho * wo, cblk)
    y_ref[...] = yb.reshape(1, n, ho, wo, cblk)
    avg_ref[...] = jnp.mean(y3, axis=1).reshape(1, n, 1, cblk)
    max_ref[...] = jnp.max(y3, axis=1).reshape(1, n, 1, cblk)


def _k3_body(y_ref, res_ref, avg_ref, max_ref, ca1_ref, ca2_ref,
             sa_a_ref, sa_m_ref, o_ref, apad_ref, mpad_ref):
    _, ho, wo, c = o_ref.shape

    avg = jnp.concatenate([avg_ref[0, 0, 0], avg_ref[1, 0, 0]], axis=-1)
    mx = jnp.concatenate([max_ref[0, 0, 0], max_ref[1, 0, 0]], axis=-1)
    v = jnp.stack([avg, mx], axis=0)
    hmid = jnp.maximum(jnp.dot(v, ca1_ref[...],
                               preferred_element_type=jnp.float32), 0.0)
    o2 = jnp.dot(hmid, ca2_ref[...], preferred_element_type=jnp.float32)
    att = jax.nn.sigmoid(o2[0] + o2[1])

    yv = jnp.concatenate([y_ref[0, 0], y_ref[1, 0]], axis=-1)
    u = yv * att[None, None, :]

    apad_ref[...] = jnp.zeros(apad_ref.shape, jnp.float32)
    mpad_ref[...] = jnp.zeros(mpad_ref.shape, jnp.float32)
    apad_ref[3:3 + ho, 3:3 + wo] = jnp.mean(u, axis=-1)
    mpad_ref[3:3 + ho, 3:3 + wo] = jnp.max(u, axis=-1)

    logits = jnp.zeros((ho, wo), jnp.float32)
    for dh in range(7):
        for dw in range(7):
            logits = logits + sa_a_ref[dh, dw] * \
                apad_ref[dh:dh + ho, dw:dw + wo]
            logits = logits + sa_m_ref[dh, dw] * \
                mpad_ref[dh:dh + ho, dw:dw + wo]

    gate = jax.nn.sigmoid(logits)[:, :, None]
    rv = jnp.concatenate([res_ref[0, 0], res_ref[1, 0]], axis=-1)
    o_ref[...] = jnp.maximum(u * gate + rv, 0.0).reshape(1, ho, wo, c)


def _split_w(wt, cblk):
    kh, kw, cin, cout = wt.shape
    return wt.reshape(kh, kw, cin, 2, cblk).transpose(3, 0, 1, 2, 4) \
        .astype(jnp.bfloat16)


def kernel(x, conv1_w, conv1_b, bn1_g, bn1_b, conv2_w, conv2_b, bn2_g,
           bn2_b, ca_w1, ca_w2, sa_w, ds_w):
    eps = 1e-5
    n, cin, h, w = x.shape
    cout = conv1_w.shape[3]
    h1, w1 = h - 2, w - 2
    ho, wo = h1 - 2, w1 - 2
    cblk = cout // 2

    xh = jnp.transpose(x, (0, 2, 3, 1)).astype(jnp.bfloat16)
    w1b = _split_w(conv1_w, cblk)
    w2b = _split_w(conv2_w, cblk)
    dswb = _split_w(ds_w, cblk)

    def vec(a):
        return a.reshape(2, 1, cblk)

    sa_a = sa_w[:, :, 0, 0]
    sa_m = sa_w[:, :, 1, 0]

    def rep(shape):
        nd = len(shape)
        return pl.BlockSpec(shape, lambda i, _nd=nd: (0,) * _nd)

    def lead(shape):
        nd = len(shape)
        return pl.BlockSpec((1,) + shape[1:],
                            lambda i, _nd=nd: (i,) + (0,) * (_nd - 1))

    k1_flops = 2 * n * h1 * w1 * 9 * cin * cout \
        + 2 * n * ho * wo * 25 * cin * cout
    t1, res = pl.pallas_call(
        functools.partial(_k1_body, eps=eps),
        out_shape=(
            jax.ShapeDtypeStruct((2, n, h1, w1, cblk), jnp.bfloat16),
            jax.ShapeDtypeStruct((2, n, ho, wo, cblk), jnp.float32)),
        grid=(2,),
        in_specs=[rep(xh.shape), lead(w1b.shape),
                  lead((2, 1, cblk)), lead((2, 1, cblk)), lead((2, 1, cblk)),
                  lead(dswb.shape)],
        out_specs=(lead((2, n, h1, w1, cblk)), lead((2, n, ho, wo, cblk))),
        compiler_params=pltpu.CompilerParams(
            dimension_semantics=("parallel",),
            vmem_limit_bytes=_VMEM_LIMIT),
        cost_estimate=pl.CostEstimate(
            flops=int(k1_flops), transcendentals=int(cout),
            bytes_accessed=int(2 * (xh.size + w1b.size + dswb.size)
                               + 2 * n * h1 * w1 * cout
                               + 4 * n * ho * wo * cout)),
    )(xh, w1b, vec(conv1_b), vec(bn1_g), vec(bn1_b), dswb)

    k2_flops = 2 * n * ho * wo * 9 * cout * cout
    y, avgp, maxp = pl.pallas_call(
        functools.partial(_k2_body, eps=eps),
        out_shape=(
            jax.ShapeDtypeStruct((2, n, ho, wo, cblk), jnp.float32),
            jax.ShapeDtypeStruct((2, n, 1, cblk), jnp.float32),
            jax.ShapeDtypeStruct((2, n, 1, cblk), jnp.float32)),
        grid=(2,),
        in_specs=[rep(t1.shape), lead(w2b.shape),
                  lead((2, 1, cblk)), lead((2, 1, cblk)), lead((2, 1, cblk))],
        out_specs=(lead((2, n, ho, wo, cblk)),
                   lead((2, n, 1, cblk)), lead((2, n, 1, cblk))),
        compiler_params=pltpu.CompilerParams(
            dimension_semantics=("parallel",),
            vmem_limit_bytes=_VMEM_LIMIT),
        cost_estimate=pl.CostEstimate(
            flops=int(k2_flops), transcendentals=int(cout),
            bytes_accessed=int(2 * (t1.size + w2b.size)
                               + 4 * n * ho * wo * cout)),
    )(t1, w2b, vec(conv2_b), vec(bn2_g), vec(bn2_b))

    def per_n(shape):
        nd = len(shape)
        blk = (shape[0], 1) + shape[2:]
        return pl.BlockSpec(blk, lambda j, _nd=nd: (0, j) + (0,) * (_nd - 2))

    out = pl.pallas_call(
        _k3_body,
        out_shape=jax.ShapeDtypeStruct((n, ho, wo, cout), jnp.float32),
        grid=(n,),
        in_specs=[per_n((2, n, ho, wo, cblk)),
                  per_n((2, n, ho, wo, cblk)),
                  per_n((2, n, 1, cblk)), per_n((2, n, 1, cblk)),
                  rep(ca_w1.shape), rep(ca_w2.shape),
                  pl.BlockSpec(memory_space=pltpu.MemorySpace.SMEM),
                  pl.BlockSpec(memory_space=pltpu.MemorySpace.SMEM)],
        out_specs=pl.BlockSpec((1, ho, wo, cout), lambda j: (j, 0, 0, 0)),
        scratch_shapes=[pltpu.VMEM((ho + 6, wo + 6), jnp.float32),
                        pltpu.VMEM((ho + 6, wo + 6), jnp.float32)],
        compiler_params=pltpu.CompilerParams(
            dimension_semantics=("parallel",),
            vmem_limit_bytes=_VMEM_LIMIT),
        cost_estimate=pl.CostEstimate(
            flops=int(20 * n * ho * wo * cout),
            transcendentals=int(n * (ho * wo + 2 * cout)),
            bytes_accessed=int(4 * (3 * n * ho * wo * cout))),
    )(y, res, avgp, maxp, ca_w1, ca_w2, sa_a, sa_m)

    return jnp.transpose(out, (0, 3, 1, 2))


# speedup vs baseline: 4.8481x; 2.4836x over previous
import functools

import jax
import jax.numpy as jnp
from jax.experimental import pallas as pl
from jax.experimental.pallas import tpu as pltpu

_VMEM_LIMIT = 48 * 1024 * 1024


def _conv_acc(xv, wv, ho, wo, kh, kw):
    n = xv.shape[0]
    cin = xv.shape[3]
    cblk = wv.shape[3]
    acc = jnp.zeros((n * ho * wo, cblk), jnp.float32)
    for dh in range(kh):
        for dw in range(kw):
            lhs = xv[:, dh:dh + ho, dw:dw + wo, :].reshape(n * ho * wo, cin)
            acc = acc + jnp.dot(lhs, wv[dh, dw],
                                preferred_element_type=jnp.float32)
    return acc


def _bn_affine(y, g, be, cnt, eps):
    mean = jnp.sum(y, axis=0) / cnt
    var = jnp.sum(y * y, axis=0) / cnt - mean * mean
    scale = g * jax.lax.rsqrt(var + eps)
    shift = be - mean * scale
    return y * scale + shift


def _k1_body(x_ref, w1_ref, b1_ref, g1_ref, be1_ref, dsw_ref,
             t1_ref, res_ref, *, eps):
    n, h1, w1o, cblk = t1_ref.shape
    ho, wo = res_ref.shape[1], res_ref.shape[2]
    xv = x_ref[...].astype(jnp.bfloat16)

    w1 = w1_ref[...].astype(jnp.bfloat16)
    y = _conv_acc(xv, w1, h1, w1o, 3, 3) + b1_ref[0]
    t1 = jnp.maximum(
        _bn_affine(y, g1_ref[0], be1_ref[0], float(n * h1 * w1o), eps), 0.0)
    t1_ref[...] = t1.reshape(n, h1, w1o, cblk).astype(t1_ref.dtype)

    dsw = dsw_ref[...].astype(jnp.bfloat16)
    res = _conv_acc(xv, dsw, ho, wo, 5, 5)
    res_ref[...] = res.reshape(n, ho, wo, cblk)


def _k2_body(t1_ref, w2_ref, b2_ref, g2_ref, be2_ref,
             y_ref, avg_ref, max_ref, *, eps):
    n, ho, wo, cblk = y_ref.shape
    tv = t1_ref[...]
    w2 = w2_ref[...].astype(jnp.bfloat16)

    y = _conv_acc(tv, w2, ho, wo, 3, 3) + b2_ref[0]
    yb = _bn_affine(y, g2_ref[0], be2_ref[0], float(n * ho * wo), eps)

    y3 = yb.reshape(n, ho * wo, cblk)
    y_ref[...] = yb.reshape(n, ho, wo, cblk)
    avg_ref[...] = jnp.mean(y3, axis=1)
    max_ref[...] = jnp.max(y3, axis=1)


def _k3_body(y_ref, res_ref, avg_ref, max_ref, ca1_ref, ca2_ref,
             sa_a_ref, sa_m_ref, o_ref, apad_ref, mpad_ref):
    _, ho, wo, c = o_ref.shape

    nb = avg_ref.shape[0]
    v = jnp.concatenate([avg_ref[...], max_ref[...]], axis=0)
    hmid = jnp.maximum(jnp.dot(v, ca1_ref[...],
                               preferred_element_type=jnp.float32), 0.0)
    o2 = jnp.dot(hmid, ca2_ref[...], preferred_element_type=jnp.float32)
    att_all = jax.nn.sigmoid(o2[:nb] + o2[nb:])
    sel = (jax.lax.broadcasted_iota(jnp.int32, (nb, 1), 0)
           == pl.program_id(0)).astype(jnp.float32)
    att = jnp.sum(att_all * sel, axis=0)

    u = y_ref[0] * att[None, None, :]

    apad_ref[...] = jnp.zeros(apad_ref.shape, jnp.float32)
    mpad_ref[...] = jnp.zeros(mpad_ref.shape, jnp.float32)
    apad_ref[3:3 + ho, 3:3 + wo] = jnp.mean(u, axis=-1)
    mpad_ref[3:3 + ho, 3:3 + wo] = jnp.max(u, axis=-1)

    logits = jnp.zeros((ho, wo), jnp.float32)
    for dh in range(7):
        for dw in range(7):
            logits = logits + sa_a_ref[dh, dw] * \
                apad_ref[dh:dh + ho, dw:dw + wo]
            logits = logits + sa_m_ref[dh, dw] * \
                mpad_ref[dh:dh + ho, dw:dw + wo]

    gate = jax.nn.sigmoid(logits)[:, :, None]
    o_ref[...] = jnp.maximum(u * gate + res_ref[0], 0.0) \
        .reshape(1, ho, wo, c)


def kernel(x, conv1_w, conv1_b, bn1_g, bn1_b, conv2_w, conv2_b, bn2_g,
           bn2_b, ca_w1, ca_w2, sa_w, ds_w):
    eps = 1e-5
    n, cin, h, w = x.shape
    cout = conv1_w.shape[3]
    h1, w1 = h - 2, w - 2
    ho, wo = h1 - 2, w1 - 2
    cblk = min(128, cout)
    nblk = cout // cblk

    xh = jnp.transpose(x, (0, 2, 3, 1)).astype(jnp.bfloat16)
    sa_a = sa_w[:, :, 0, 0]
    sa_m = sa_w[:, :, 1, 0]

    def rep(shape):
        nd = len(shape)
        return pl.BlockSpec(shape, lambda i, _nd=nd: (0,) * _nd)

    def wspec(shape):
        return pl.BlockSpec(shape[:3] + (cblk,), lambda i: (0, 0, 0, i))

    vspec = pl.BlockSpec((1, cblk), lambda i: (0, i))

    k1_flops = 2 * n * h1 * w1 * 9 * cin * cout \
        + 2 * n * ho * wo * 25 * cin * cout
    t1, res = pl.pallas_call(
        functools.partial(_k1_body, eps=eps),
        out_shape=(
            jax.ShapeDtypeStruct((n, h1, w1, cout), jnp.bfloat16),
            jax.ShapeDtypeStruct((n, ho, wo, cout), jnp.float32)),
        grid=(nblk,),
        in_specs=[rep(xh.shape), wspec(conv1_w.shape),
                  vspec, vspec, vspec, wspec(ds_w.shape)],
        out_specs=(pl.BlockSpec((n, h1, w1, cblk), lambda i: (0, 0, 0, i)),
                   pl.BlockSpec((n, ho, wo, cblk), lambda i: (0, 0, 0, i))),
        compiler_params=pltpu.CompilerParams(
            dimension_semantics=("parallel",),
            vmem_limit_bytes=_VMEM_LIMIT),
        cost_estimate=pl.CostEstimate(
            flops=int(k1_flops), transcendentals=int(cout),
            bytes_accessed=int(2 * xh.size + 4 * conv1_w.size
                               + 4 * ds_w.size + 2 * n * h1 * w1 * cout
                               + 4 * n * ho * wo * cout)),
    )(xh, conv1_w, conv1_b.reshape(1, cout), bn1_g.reshape(1, cout),
      bn1_b.reshape(1, cout), ds_w)

    k2_flops = 2 * n * ho * wo * 9 * cout * cout
    y, avgp, maxp = pl.pallas_call(
        functools.partial(_k2_body, eps=eps),
        out_shape=(
            jax.ShapeDtypeStruct((n, ho, wo, cout), jnp.float32),
            jax.ShapeDtypeStruct((n, cout), jnp.float32),
            jax.ShapeDtypeStruct((n, cout), jnp.float32)),
        grid=(nblk,),
        in_specs=[rep(t1.shape), wspec(conv2_w.shape),
                  vspec, vspec, vspec],
        out_specs=(pl.BlockSpec((n, ho, wo, cblk), lambda i: (0, 0, 0, i)),
                   pl.BlockSpec((n, cblk), lambda i: (0, i)),
                   pl.BlockSpec((n, cblk), lambda i: (0, i))),
        compiler_params=pltpu.CompilerParams(
            dimension_semantics=("parallel",),
            vmem_limit_bytes=_VMEM_LIMIT),
        cost_estimate=pl.CostEstimate(
            flops=int(k2_flops), transcendentals=int(cout),
            bytes_accessed=int(2 * t1.size + 4 * conv2_w.size
                               + 4 * n * ho * wo * cout)),
    )(t1, conv2_w, conv2_b.reshape(1, cout), bn2_g.reshape(1, cout),
      bn2_b.reshape(1, cout))

    def per_n(shape):
        nd = len(shape)
        return pl.BlockSpec((1,) + shape[1:],
                            lambda j, _nd=nd: (j,) + (0,) * (_nd - 1))

    out = pl.pallas_call(
        _k3_body,
        out_shape=jax.ShapeDtypeStruct((n, ho, wo, cout), jnp.float32),
        grid=(n,),
        in_specs=[per_n((n, ho, wo, cout)), per_n((n, ho, wo, cout)),
                  rep((n, cout)), rep((n, cout)),
                  rep(ca_w1.shape), rep(ca_w2.shape),
                  pl.BlockSpec(memory_space=pltpu.MemorySpace.SMEM),
                  pl.BlockSpec(memory_space=pltpu.MemorySpace.SMEM)],
        out_specs=per_n((n, ho, wo, cout)),
        scratch_shapes=[pltpu.VMEM((ho + 6, wo + 6), jnp.float32),
                        pltpu.VMEM((ho + 6, wo + 6), jnp.float32)],
        compiler_params=pltpu.CompilerParams(
            dimension_semantics=("parallel",),
            vmem_limit_bytes=_VMEM_LIMIT),
        cost_estimate=pl.CostEstimate(
            flops=int(20 * n * ho * wo * cout),
            transcendentals=int(n * (ho * wo + 2 * cout)),
            bytes_accessed=int(4 * (3 * n * ho * wo * cout))),
    )(y, res, avgp, maxp, ca_w1, ca_w2, sa_a, sa_m)

    return jnp.transpose(out, (0, 3, 1, 2))
